# baseline (speedup 1.0000x reference)
import jax
import jax.numpy as jnp
from jax import lax
from jax.experimental import pallas as pl
from jax.experimental.pallas import tpu as pltpu

_NEG_INF = float("-inf")
_BIG = 2147483647
_RPB = 8


def _beam_step_kernel(lp40_ref, sums_ref, seq_ref, lp_ref, bsl_ref, st_ref,
                      seq_out_ref, ys_out_ref, bsl_out_ref, st_out_ref,
                      scr_ref):
    R = lp40_ref.shape[1]
    C = lp40_ref.shape[2]
    K = R // _RPB
    T = bsl_ref.shape[2]
    cand = lp40_ref[0] + sums_ref[0]
    scr_ref[...] = cand
    rowmax = jnp.max(cand, axis=1, keepdims=True)
    riota = lax.broadcasted_iota(jnp.int32, (R, 1), 0)
    ciota = lax.broadcasted_iota(jnp.int32, (1, C), 1)
    i8 = lax.broadcasted_iota(jnp.int32, (1, 8), 1)
    i16 = lax.broadcasted_iota(jnp.int32, (1, 16), 1)
    i10 = lax.broadcasted_iota(jnp.int32, (1, K * T), 1)
    seq_row = seq_ref[0]
    ys_row = jnp.zeros((1, 8), jnp.float32)
    seq_out_row = jnp.zeros((1, 16), jnp.int32)
    for k in range(K):
        m = jnp.max(rowmax)
        r = jnp.min(jnp.where(rowmax == m, riota, _BIG))
        row = scr_ref[pl.ds(r, 1), :]
        c = jnp.min(jnp.where(row == m, ciota, _BIG))
        bix = r // _RPB
        six = (r - bix * _RPB) * C + c
        masked = jnp.where(ciota == c, _NEG_INF, row)
        scr_ref[pl.ds(r, 1), :] = masked
        rowmax = jnp.where(riota == r, jnp.max(masked), rowmax)
        ys_row = jnp.where(i8 == k, m, ys_row)
        for t in range(T):
            val = jnp.sum(jnp.where(i10 == bix * T + t, seq_row, 0))
            seq_out_row = jnp.where(i16 == k * (T + 1) + t, val, seq_out_row)
        seq_out_row = jnp.where(i16 == k * (T + 1) + T, six, seq_out_row)
        bsl_out_ref[0, pl.ds(k, 1), pl.ds(0, T), :] = (
            bsl_ref[0, pl.ds(bix, 1), :, :])
        bsl_out_ref[0, pl.ds(k, 1), pl.ds(T, 1), :] = (
            lp_ref[pl.ds(0, 1), pl.ds(bix, 1), :])
        st_out_ref[:, 0, pl.ds(k, 1), :] = st_ref[:, 0, pl.ds(bix, 1), :]
    ys_out_ref[0] = ys_row
    seq_out_ref[0] = seq_out_row


def kernel(logprobs, beam_logprobs_sum, beam_seq, beam_seq_logprobs, state,
           beam_size):
    B, K = beam_logprobs_sum.shape
    V = logprobs.shape[-1]
    T = beam_seq.shape[-1]
    S, BK, D = state.shape
    R = K * _RPB
    C = V // _RPB

    lp40 = logprobs.reshape(B, R, C)
    sums40 = jnp.broadcast_to(beam_logprobs_sum[:, :, None],
                              (B, K, _RPB)).reshape(B, R, 1)
    seq3 = beam_seq.reshape(B, 1, K * T)
    lp3 = logprobs.reshape(B, K, V)
    st4 = state.reshape(S, B, K, D)

    out_shapes = (
        jax.ShapeDtypeStruct((B, 1, 16), jnp.int32),
        jax.ShapeDtypeStruct((B, 1, 8), jnp.float32),
        jax.ShapeDtypeStruct((B, K, T + 1, V), jnp.float32),
        jax.ShapeDtypeStruct((S, B, K, D), jnp.float32),
    )
    seq_out, ys_out, bsl_out, st_out = pl.pallas_call(
        _beam_step_kernel,
        grid=(B,),
        in_specs=[
            pl.BlockSpec((1, R, C), lambda b: (b, 0, 0)),
            pl.BlockSpec((1, R, 1), lambda b: (b, 0, 0)),
            pl.BlockSpec((1, 1, K * T), lambda b: (b, 0, 0)),
            pl.BlockSpec((1, K, V), lambda b: (b, 0, 0)),
            pl.BlockSpec((1, K, T, V), lambda b: (b, 0, 0, 0)),
            pl.BlockSpec((S, 1, K, D), lambda b: (0, b, 0, 0)),
        ],
        out_specs=[
            pl.BlockSpec((1, 1, 16), lambda b: (b, 0, 0)),
            pl.BlockSpec((1, 1, 8), lambda b: (b, 0, 0)),
            pl.BlockSpec((1, K, T + 1, V), lambda b: (b, 0, 0, 0)),
            pl.BlockSpec((S, 1, K, D), lambda b: (0, b, 0, 0)),
        ],
        out_shape=out_shapes,
        scratch_shapes=[pltpu.VMEM((R, C), jnp.float32)],
        compiler_params=pltpu.CompilerParams(
            vmem_limit_bytes=128 * 1024 * 1024),
    )(lp40, sums40, seq3, lp3, beam_seq_logprobs, st4)

    new_beam_seq = seq_out[:, 0, :K * (T + 1)].reshape(B, K, T + 1)
    new_beam_logprobs_sum = ys_out[:, 0, :K]
    new_state = st_out.reshape(S, B * K, D)
    return (new_beam_seq, bsl_out, new_beam_logprobs_sum, new_state)
